# baseline (speedup 1.0000x reference)
import jax
import jax.numpy as jnp
from jax.experimental import pallas as pl


def _add_body(x_ref, w_ref, o_ref):
    o_ref[...] = x_ref[...] + w_ref[...]


def kernel(x, weight):
    B, S, D = x.shape
    w = weight[:S]
    TS = 256
    grid = (S // TS, B)
    return pl.pallas_call(
        _add_body,
        grid=grid,
        in_specs=[
            pl.BlockSpec((1, TS, D), lambda s, b: (b, s, 0)),
            pl.BlockSpec((TS, D), lambda s, b: (s, 0)),
        ],
        out_specs=pl.BlockSpec((1, TS, D), lambda s, b: (b, s, 0)),
        out_shape=jax.ShapeDtypeStruct((B, S, D), x.dtype),
    )(x, w)


# speedup vs baseline: 1.4272x; 1.4272x over previous
import jax
import jax.numpy as jnp
from jax.experimental import pallas as pl


def _add_body(x_ref, w_ref, o_ref):
    o_ref[...] = x_ref[...] + w_ref[...]


def kernel(x, weight):
    B, S, D = x.shape
    w = weight[:S]
    TS = 1024
    grid = (S // TS, B)
    return pl.pallas_call(
        _add_body,
        grid=grid,
        in_specs=[
            pl.BlockSpec((1, TS, D), lambda s, b: (b, s, 0)),
            pl.BlockSpec((TS, D), lambda s, b: (s, 0)),
        ],
        out_specs=pl.BlockSpec((1, TS, D), lambda s, b: (b, s, 0)),
        out_shape=jax.ShapeDtypeStruct((B, S, D), x.dtype),
    )(x, w)


# speedup vs baseline: 1.5508x; 1.0866x over previous
import jax
import jax.numpy as jnp
from jax.experimental import pallas as pl


def _add_body(x_ref, w_ref, o_ref):
    o_ref[...] = x_ref[...] + w_ref[...]


def kernel(x, weight):
    B, S, D = x.shape
    w = weight[:S]
    TS = 2048
    grid = (S // TS, B)
    return pl.pallas_call(
        _add_body,
        grid=grid,
        in_specs=[
            pl.BlockSpec((1, TS, D), lambda s, b: (b, s, 0)),
            pl.BlockSpec((TS, D), lambda s, b: (s, 0)),
        ],
        out_specs=pl.BlockSpec((1, TS, D), lambda s, b: (b, s, 0)),
        out_shape=jax.ShapeDtypeStruct((B, S, D), x.dtype),
    )(x, w)


# speedup vs baseline: 1.5742x; 1.0151x over previous
import jax
import jax.numpy as jnp
from jax.experimental import pallas as pl
from jax.experimental.pallas import tpu as pltpu


def _body(x_hbm, w_hbm, o_hbm, xb, wb, ob, sem_w, sem_x, sem_o):
    B = x_hbm.shape[0]

    def x_in(i, slot):
        return pltpu.make_async_copy(x_hbm.at[i], xb.at[slot], sem_x.at[slot])

    def o_out(i, slot):
        return pltpu.make_async_copy(ob.at[slot], o_hbm.at[i], sem_o.at[slot])

    pltpu.make_async_copy(w_hbm, wb, sem_w).start()
    x_in(0, 0).start()
    x_in(1, 1).start()
    pltpu.make_async_copy(w_hbm, wb, sem_w).wait()

    for i in range(B):
        slot = i % 2
        x_in(i, slot).wait()
        if i >= 2:
            o_out(i - 2, slot).wait()
        ob[slot] = xb[slot] + wb[...]
        o_out(i, slot).start()
        if i + 2 < B:
            x_in(i + 2, slot).start()

    o_out(B - 2, (B - 2) % 2).wait()
    o_out(B - 1, (B - 1) % 2).wait()


def kernel(x, weight):
    B, S, D = x.shape
    w = weight[:S]
    return pl.pallas_call(
        _body,
        in_specs=[
            pl.BlockSpec(memory_space=pl.ANY),
            pl.BlockSpec(memory_space=pl.ANY),
        ],
        out_specs=pl.BlockSpec(memory_space=pl.ANY),
        out_shape=jax.ShapeDtypeStruct((B, S, D), x.dtype),
        scratch_shapes=[
            pltpu.VMEM((2, S, D), x.dtype),
            pltpu.VMEM((S, D), x.dtype),
            pltpu.VMEM((2, S, D), x.dtype),
            pltpu.SemaphoreType.DMA,
            pltpu.SemaphoreType.DMA((2,)),
            pltpu.SemaphoreType.DMA((2,)),
        ],
        compiler_params=pltpu.CompilerParams(vmem_limit_bytes=56 * 1024 * 1024),
    )(x, w)
